# baseline (speedup 1.0000x reference)
import jax
import jax.numpy as jnp
from jax import lax
from jax.experimental import pallas as pl
from jax.experimental.pallas import tpu as pltpu
from jax.experimental.pallas import tpu_sc as plsc

B = 16384
C = 128
NC = 2
NS = 16
NW = NC * NS
BPW = B // NW
WCH = 128
NCHUNK = BPW // WCH
GPC = WCH // 16


def _sc_body(y_hbm, t_hbm, nan_hbm, out_hbm, y_v, rows_v, tbl_v, ysem, sem, wsem):
    sid = lax.axis_index("s")
    wid = sid * NC + lax.axis_index("c")
    base = wid * BPW
    y_copy = pltpu.async_copy(y_hbm.at[pl.ds(base, BPW)], y_v, ysem)
    t_copy = pltpu.async_copy(t_hbm, tbl_v.at[pl.ds(0, 2 * C)], sem)
    n_copy = pltpu.async_copy(nan_hbm, tbl_v.at[pl.ds(2 * C, C)], sem)
    t_copy.wait()
    n_copy.wait()
    y_copy.wait()

    t0 = tuple(tbl_v[pl.ds(j * 16, 16)] for j in range(8))
    t1 = tuple(tbl_v[pl.ds(C + j * 16, 16)] for j in range(8))
    t2 = tuple(tbl_v[pl.ds(2 * C + j * 16, 16)] for j in range(8))
    d1 = tuple(t1[j] - t0[j] for j in range(8))
    d2 = tuple(t2[j] - t0[j] for j in range(8))
    carry0 = (*t0, *d1, *d2)

    dnums = lax.GatherDimensionNumbers(
        offset_dims=(), collapsed_slice_dims=(0,), start_index_map=(0,))

    def _grp(g, carry):
        ct0, cd1, cd2 = carry[0:8], carry[8:16], carry[16:24]
        v = y_v[pl.ds(g * 16, 16)]
        cls = jnp.where(v == v, v, jnp.float32(2.0))
        gbase = g * 16 * C
        for r in range(16):
            c = lax.gather(cls, jnp.full((16, 1), r, jnp.int32), dnums, (1,),
                           mode=lax.GatherScatterMode.PROMISE_IN_BOUNDS)
            a = jnp.where(c == 1.0, jnp.float32(1.0), jnp.float32(0.0))
            b = jnp.where(c == 2.0, jnp.float32(1.0), jnp.float32(0.0))
            for j in range(8):
                rows_v[pl.ds(gbase + r * C + j * 16, 16)] = (
                    ct0[j] + a * cd1[j] + b * cd2[j])
        return carry

    writes = []
    for ch in range(NCHUNK):
        lax.fori_loop(ch * GPC, (ch + 1) * GPC, _grp, carry0)
        writes.append(
            pltpu.async_copy(
                rows_v.at[pl.ds(ch * WCH * C, WCH * C)],
                out_hbm.at[pl.ds((base + ch * WCH) * C, WCH * C)],
                wsem,
            )
        )
    for w in writes:
        w.wait()


@jax.jit
def _label_embed_sc(y, t_flat, nan_flat):
    mesh = plsc.VectorSubcoreMesh(core_axis_name="c", subcore_axis_name="s")
    f = pl.kernel(
        _sc_body,
        out_type=jax.ShapeDtypeStruct((B * C,), jnp.float32),
        mesh=mesh,
        scratch_types=[
            pltpu.VMEM((BPW,), jnp.float32),
            pltpu.VMEM((BPW * C,), jnp.float32),
            pltpu.VMEM((3 * C,), jnp.float32),
            pltpu.SemaphoreType.DMA,
            pltpu.SemaphoreType.DMA,
            pltpu.SemaphoreType.DMA,
        ],
    )
    return f(y, t_flat, nan_flat)


def kernel(y, table, nan_emb):
    out = _label_embed_sc(y, table.reshape(-1), nan_emb)
    return out.reshape(B, C)


# speedup vs baseline: 1.2336x; 1.2336x over previous
import jax
import jax.numpy as jnp
from jax import lax
from jax.experimental import pallas as pl
from jax.experimental.pallas import tpu as pltpu
from jax.experimental.pallas import tpu_sc as plsc

B = 16384
C = 128
NC = 2
NS = 16
NW = NC * NS
BPW = B // NW
IDXW = 128
NCHUNK = BPW // IDXW
NGRP = BPW // 16
GPC = IDXW // 16


def _sc_body(y_hbm, t_hbm, nan_hbm, out_hbm, y_v, idx_v, rows_v, tbl_sh,
             ysem, sem, wsem):
    sid = lax.axis_index("s")
    wid = sid * NC + lax.axis_index("c")
    base = wid * BPW
    y_copy = pltpu.async_copy(y_hbm.at[pl.ds(base, BPW)], y_v, ysem)
    t_copy = pltpu.async_copy(t_hbm, tbl_sh.at[pl.ds(sid * 3, 2)], sem)
    n_copy = pltpu.async_copy(nan_hbm, tbl_sh.at[pl.ds(sid * 3 + 2, 1)], sem)
    t_copy.wait()
    n_copy.wait()
    y_copy.wait()
    row_off = jnp.full((16,), sid * 3, jnp.int32)
    gathers = []
    for j in range(NCHUNK):
        for gg in range(GPC):
            g = j * GPC + gg
            v = y_v[pl.ds(g * 16, 16)]
            not_nan = v == v
            idx = jnp.where(not_nan, v, jnp.float32(2.0)).astype(jnp.int32)
            idx_v[j, pl.ds(gg * 16, 16)] = idx + row_off
        gathers.append(
            pltpu.async_copy(
                tbl_sh.at[idx_v.at[j]],
                rows_v.at[pl.ds(j * IDXW, IDXW)],
                sem,
            )
        )
    writes = []
    for j in range(NCHUNK):
        gathers[j].wait()
        writes.append(
            pltpu.async_copy(
                rows_v.at[pl.ds(j * IDXW, IDXW)],
                out_hbm.at[pl.ds(base + j * IDXW, IDXW)],
                wsem,
            )
        )
    for w in writes:
        w.wait()


@jax.jit
def _label_embed_sc(y, table, nan2d):
    mesh = plsc.VectorSubcoreMesh(core_axis_name="c", subcore_axis_name="s")
    f = pl.kernel(
        _sc_body,
        out_type=jax.ShapeDtypeStruct((B, C), jnp.float32),
        mesh=mesh,
        scratch_types=[
            pltpu.VMEM((BPW,), jnp.float32),
            pltpu.VMEM((NCHUNK, IDXW), jnp.int32),
            pltpu.VMEM((BPW, C), jnp.float32),
            pltpu.VMEM_SHARED((NS * 3, C), jnp.float32),
            pltpu.SemaphoreType.DMA,
            pltpu.SemaphoreType.DMA,
            pltpu.SemaphoreType.DMA,
        ],
    )
    return f(y, table, nan2d)


def kernel(y, table, nan_emb):
    return _label_embed_sc(y, table, nan_emb[None, :])
